# baseline (speedup 1.0000x reference)
import functools

import jax
import jax.numpy as jnp
from jax import lax
from jax.experimental import pallas as pl
from jax.experimental.pallas import tpu as pltpu
from jax.experimental.pallas import tpu_sc as plsc

B, S, D, F, E = 4, 2048, 1024, 4096, 8
N = B * S
TM = 256
TMS = TM.bit_length() - 1
NP = N + E * TM
NB = NP // TM
LANES = 128

RB = 1024
NRB = N // RB
SB = 1024
NSB = N // SB


def _router_body(x_ref, wr_ref, br_ref, rank_ref, eid_ref, off_ref, be_ref,
                 counts_scr):
    i = pl.program_id(0)

    @pl.when(i == 0)
    def _init():
        counts_scr[...] = jnp.zeros((1, LANES), jnp.float32)

    @pl.when(i < NRB)
    def _sweep1():
        xb = x_ref[...]
        scores = jnp.dot(xb, wr_ref[...],
                         preferred_element_type=jnp.float32,
                         precision=lax.Precision.DEFAULT) + br_ref[...]
        m = jnp.max(scores, axis=1, keepdims=True)
        lane = lax.broadcasted_iota(jnp.int32, (RB, LANES), 1)
        idx = jnp.min(jnp.where(scores == m, lane, LANES - 1),
                      axis=1, keepdims=True)
        onehot = (lane == idx).astype(jnp.float32)
        row = lax.broadcasted_iota(jnp.int32, (RB, RB), 0)
        col = lax.broadcasted_iota(jnp.int32, (RB, RB), 1)
        tri = (col <= row).astype(jnp.float32)
        pre = jnp.dot(tri, onehot,
                      preferred_element_type=jnp.float32,
                      precision=lax.Precision.HIGHEST)
        carry = counts_scr[...]
        rank = jnp.sum(onehot * (carry + pre - 1.0), axis=1, keepdims=True)
        rank_ref[...] = rank.astype(jnp.int32)
        eid_ref[...] = idx
        counts_scr[...] = carry + pre[RB - 1:RB, :]

    @pl.when(i == NRB)
    def _offsets():
        counts = counts_scr[...]
        ci = counts.astype(jnp.int32)
        pci = ((ci + (TM - 1)) >> TMS) << TMS
        pcf = pci.astype(jnp.float32)
        r0 = lax.broadcasted_iota(jnp.int32, (LANES, LANES), 0)
        c0 = lax.broadcasted_iota(jnp.int32, (LANES, LANES), 1)
        tri_strict = (r0 < c0).astype(jnp.float32)
        off = jnp.dot(pcf, tri_strict,
                      preferred_element_type=jnp.float32,
                      precision=lax.Precision.HIGHEST)
        off_col = jnp.sum((c0 < r0).astype(jnp.float32) * pcf,
                          axis=1, keepdims=True)
        off_ref[...] = off_col.astype(jnp.int32)
        lane1 = lax.broadcasted_iota(jnp.int32, (1, LANES), 1)
        end = jnp.where(lane1 < E, off + pcf, 3.0e38)
        brow = lax.broadcasted_iota(
            jnp.int32, (LANES, LANES), 0).astype(jnp.float32) * float(TM)
        cmp = (brow >= end).astype(jnp.float32)
        bevec = jnp.sum(cmp, axis=1, keepdims=True)
        be_ref[...] = jnp.minimum(bevec, float(E - 1)).astype(jnp.int32)


def _run_router(x2, wrp, brp):
    return pl.pallas_call(
        _router_body,
        grid=(NRB + 1,),
        in_specs=[
            pl.BlockSpec((RB, D), lambda i: (jnp.minimum(i, NRB - 1), 0)),
            pl.BlockSpec((D, LANES), lambda i: (0, 0)),
            pl.BlockSpec((1, LANES), lambda i: (0, 0)),
        ],
        out_specs=[
            pl.BlockSpec((RB, 1), lambda i: (jnp.minimum(i, NRB - 1), 0)),
            pl.BlockSpec((RB, 1), lambda i: (jnp.minimum(i, NRB - 1), 0)),
            pl.BlockSpec((LANES, 1), lambda i: (0, 0)),
            pl.BlockSpec((LANES, 1), lambda i: (0, 0)),
        ],
        out_shape=[
            jax.ShapeDtypeStruct((N, 1), jnp.int32),
            jax.ShapeDtypeStruct((N, 1), jnp.int32),
            jax.ShapeDtypeStruct((LANES, 1), jnp.int32),
            jax.ShapeDtypeStruct((LANES, 1), jnp.int32),
        ],
        scratch_shapes=[
            pltpu.VMEM((1, LANES), jnp.float32),
        ],
    )(x2, wrp, brp)


_INFO = plsc.get_sparse_core_info()
_NC, _NS = _INFO.num_cores, _INFO.num_subcores
NW = _NC * _NS
TPW = N // NW
CH = 64

_MESH = plsc.VectorSubcoreMesh(core_axis_name="c", subcore_axis_name="s")


def _slots_chunk(rank_hbm, eid_hbm, base, rank_v, eid_v, off_v, idx_v):
    pltpu.sync_copy(rank_hbm.at[pl.ds(base, CH)], rank_v)
    pltpu.sync_copy(eid_hbm.at[pl.ds(base, CH)], eid_v)
    ov = off_v[...]
    for k in range(CH // 16):
        ev = eid_v[pl.ds(k * 16, 16)]
        rv = rank_v[pl.ds(k * 16, 16)]
        og = lax.gather(
            ov, ev[:, None],
            dimension_numbers=lax.GatherDimensionNumbers(
                offset_dims=(), collapsed_slice_dims=(0,),
                start_index_map=(0,)),
            slice_sizes=(1,),
            mode=lax.GatherScatterMode.PROMISE_IN_BOUNDS)
        idx_v[pl.ds(k * 16, 16)] = rv + og


_SC_SCRATCH = [
    pltpu.VMEM((CH,), jnp.int32),
    pltpu.VMEM((CH,), jnp.int32),
    pltpu.VMEM((CH,), jnp.int32),
    pltpu.VMEM((16,), jnp.int32),
    pltpu.VMEM((CH, D), jnp.float32),
    pltpu.SemaphoreType.DMA,
]


@functools.partial(
    pl.kernel, mesh=_MESH,
    out_type=jax.ShapeDtypeStruct((NP, D), jnp.float32),
    scratch_types=_SC_SCRATCH,
)
def _sc_dispatch(x_hbm, rank_hbm, eid_hbm, off_hbm, xs_hbm,
                 rank_v, eid_v, idx_v, off_v, rows_v, sem):
    wid = lax.axis_index("s") * _NC + lax.axis_index("c")
    pltpu.sync_copy(off_hbm, off_v)
    for j in range(TPW // CH):
        base = wid * TPW + j * CH
        _slots_chunk(rank_hbm, eid_hbm, base, rank_v, eid_v, off_v, idx_v)
        pltpu.sync_copy(x_hbm.at[pl.ds(base, CH)], rows_v)
        pltpu.async_copy(rows_v, xs_hbm.at[idx_v], sem).wait()


@functools.partial(
    pl.kernel, mesh=_MESH,
    out_type=jax.ShapeDtypeStruct((N, D), jnp.float32),
    scratch_types=_SC_SCRATCH,
)
def _sc_combine(y_hbm, rank_hbm, eid_hbm, off_hbm, out_hbm,
                rank_v, eid_v, idx_v, off_v, rows_v, sem):
    wid = lax.axis_index("s") * _NC + lax.axis_index("c")
    pltpu.sync_copy(off_hbm, off_v)
    for j in range(TPW // CH):
        base = wid * TPW + j * CH
        _slots_chunk(rank_hbm, eid_hbm, base, rank_v, eid_v, off_v, idx_v)
        pltpu.async_copy(y_hbm.at[idx_v], rows_v, sem).wait()
        pltpu.sync_copy(rows_v, out_hbm.at[pl.ds(base, CH)])


def _erf(z):
    s = jnp.where(z >= 0.0, 1.0, -1.0)
    a = jnp.abs(z)
    t = 1.0 / (1.0 + 0.3275911 * a)
    poly = t * (0.254829592 + t * (-0.284496736 + t * (
        1.421413741 + t * (-1.453152027 + t * 1.061405429))))
    return s * (1.0 - poly * jnp.exp(-a * a))


def _gelu(v):
    return 0.5 * v * (1.0 + _erf(v * 0.7071067811865476))


FT = F // 2


FC = 512


def _ffn_chunks(x_ref, w1_ref, b1_ref, w2_ref):
    h = jnp.dot(x_ref[...], w1_ref[0], preferred_element_type=jnp.float32)
    g = _gelu(h + b1_ref[0])
    return jnp.dot(g, w2_ref[0], preferred_element_type=jnp.float32)


def _ffn_body_a(be_ref, x_ref, w1_ref, b1_ref, w2_ref, y_ref):
    y_ref[...] = _ffn_chunks(x_ref, w1_ref, b1_ref, w2_ref)


def _ffn_body_b(be_ref, x_ref, w1_ref, b1_ref, w2_ref, b2_ref, y0_ref, y_ref):
    y = _ffn_chunks(x_ref, w1_ref, b1_ref, w2_ref)
    y_ref[...] = y + y0_ref[...] + b2_ref[0]


def _run_ffn(be, xs, w1, b1, w2, b2):
    def specs(half, extra_y0):
        sp = [
            pl.BlockSpec((TM, D), lambda i, be: (i, 0)),
            pl.BlockSpec((1, D, FT), lambda i, be: (be[i], 0, half)),
            pl.BlockSpec((1, 1, FT), lambda i, be: (be[i], 0, half)),
            pl.BlockSpec((1, FT, D), lambda i, be: (be[i], half, 0)),
        ]
        if extra_y0:
            sp.append(pl.BlockSpec((1, 1, D), lambda i, be: (be[i], 0, 0)))
            sp.append(pl.BlockSpec((TM, D), lambda i, be: (i, 0)))
        return sp

    def call(body, half, args, extra_y0):
        grid_spec = pltpu.PrefetchScalarGridSpec(
            num_scalar_prefetch=1,
            grid=(NB,),
            in_specs=specs(half, extra_y0),
            out_specs=pl.BlockSpec((TM, D), lambda i, be: (i, 0)),
        )
        return pl.pallas_call(
            body,
            grid_spec=grid_spec,
            out_shape=jax.ShapeDtypeStruct((NP, D), jnp.float32),
        )(be, *args)

    y0 = call(_ffn_body_a, 0, (xs, w1, b1, w2), False)
    return call(_ffn_body_b, 1, (xs, w1, b1, w2, b2, y0), True)


def kernel(x, Wr, br, W1, b1, W2, b2):
    x2 = x.reshape(N, D)
    wrp = jnp.pad(Wr, ((0, 0), (0, LANES - E)))
    brp = jnp.pad(br.reshape(1, E), ((0, 0), (0, LANES - E)),
                  constant_values=-3.0e38)
    rank2, eid2, off2, be2 = _run_router(x2, wrp, brp)
    rank = rank2.reshape(N)
    eid = eid2.reshape(N)
    off16 = off2.reshape(LANES)[:16]
    be = be2.reshape(LANES)[:NB]
    xs = _sc_dispatch(x2, rank, eid, off16)
    y = _run_ffn(be, xs, W1, b1.reshape(E, 1, F), W2, b2.reshape(E, 1, D))
    out = _sc_combine(y, rank, eid, off16)
    return out.reshape(B, S, D)


# speedup vs baseline: 1.0488x; 1.0488x over previous
import functools

import jax
import jax.numpy as jnp
from jax import lax
from jax.experimental import pallas as pl
from jax.experimental.pallas import tpu as pltpu
from jax.experimental.pallas import tpu_sc as plsc

B, S, D, F, E = 4, 2048, 1024, 4096, 8
N = B * S
TM = 256
TMS = TM.bit_length() - 1
NP = N + E * TM
NB = NP // TM
LANES = 128

RB = 1024
NRB = N // RB
SB = 1024
NSB = N // SB


def _router_body(x_ref, wr_ref, br_ref, rank_ref, eid_ref, off_ref, be_ref,
                 counts_scr):
    i = pl.program_id(0)

    @pl.when(i == 0)
    def _init():
        counts_scr[...] = jnp.zeros((1, LANES), jnp.float32)

    @pl.when(i < NRB)
    def _sweep1():
        xb = x_ref[...]
        scores = jnp.dot(xb, wr_ref[...],
                         preferred_element_type=jnp.float32,
                         precision=lax.Precision.DEFAULT) + br_ref[...]
        m = jnp.max(scores, axis=1, keepdims=True)
        lane = lax.broadcasted_iota(jnp.int32, (RB, LANES), 1)
        idx = jnp.min(jnp.where(scores == m, lane, LANES - 1),
                      axis=1, keepdims=True)
        onehot = (lane == idx).astype(jnp.float32)
        row = lax.broadcasted_iota(jnp.int32, (RB, RB), 0)
        col = lax.broadcasted_iota(jnp.int32, (RB, RB), 1)
        tri = (col <= row).astype(jnp.float32)
        pre = jnp.dot(tri, onehot,
                      preferred_element_type=jnp.float32)
        carry = counts_scr[...]
        rank = jnp.sum(onehot * (carry + pre - 1.0), axis=1, keepdims=True)
        rank_ref[...] = rank.astype(jnp.int32)
        eid_ref[...] = idx
        counts_scr[...] = carry + pre[RB - 1:RB, :]

    @pl.when(i == NRB)
    def _offsets():
        counts = counts_scr[...]
        ci = counts.astype(jnp.int32)
        pci = ((ci + (TM - 1)) >> TMS) << TMS
        pcf = pci.astype(jnp.float32)
        r0 = lax.broadcasted_iota(jnp.int32, (LANES, LANES), 0)
        c0 = lax.broadcasted_iota(jnp.int32, (LANES, LANES), 1)
        tri_strict = (r0 < c0).astype(jnp.float32)
        off = jnp.dot(pcf, tri_strict,
                      preferred_element_type=jnp.float32)
        off_col = jnp.sum((c0 < r0).astype(jnp.float32) * pcf,
                          axis=1, keepdims=True)
        off_ref[...] = off_col.astype(jnp.int32)
        lane1 = lax.broadcasted_iota(jnp.int32, (1, LANES), 1)
        end = jnp.where(lane1 < E, off + pcf, 3.0e38)
        brow = lax.broadcasted_iota(
            jnp.int32, (LANES, LANES), 0).astype(jnp.float32) * float(TM)
        cmp = (brow >= end).astype(jnp.float32)
        bevec = jnp.sum(cmp, axis=1, keepdims=True)
        be_ref[...] = jnp.minimum(bevec, float(E - 1)).astype(jnp.int32)


def _run_router(x2, wrp, brp):
    return pl.pallas_call(
        _router_body,
        grid=(NRB + 1,),
        in_specs=[
            pl.BlockSpec((RB, D), lambda i: (jnp.minimum(i, NRB - 1), 0)),
            pl.BlockSpec((D, LANES), lambda i: (0, 0)),
            pl.BlockSpec((1, LANES), lambda i: (0, 0)),
        ],
        out_specs=[
            pl.BlockSpec((RB, 1), lambda i: (jnp.minimum(i, NRB - 1), 0)),
            pl.BlockSpec((RB, 1), lambda i: (jnp.minimum(i, NRB - 1), 0)),
            pl.BlockSpec((LANES, 1), lambda i: (0, 0)),
            pl.BlockSpec((LANES, 1), lambda i: (0, 0)),
        ],
        out_shape=[
            jax.ShapeDtypeStruct((N, 1), jnp.int32),
            jax.ShapeDtypeStruct((N, 1), jnp.int32),
            jax.ShapeDtypeStruct((LANES, 1), jnp.int32),
            jax.ShapeDtypeStruct((LANES, 1), jnp.int32),
        ],
        scratch_shapes=[
            pltpu.VMEM((1, LANES), jnp.float32),
        ],
    )(x2, wrp, brp)


_INFO = plsc.get_sparse_core_info()
_NC, _NS = _INFO.num_cores, _INFO.num_subcores
NW = _NC * _NS
TPW = N // NW
CH = 64

_MESH = plsc.VectorSubcoreMesh(core_axis_name="c", subcore_axis_name="s")


def _slots_chunk(rank_hbm, eid_hbm, base, rank_v, eid_v, off_v, idx_v):
    pltpu.sync_copy(rank_hbm.at[pl.ds(base, CH)], rank_v)
    pltpu.sync_copy(eid_hbm.at[pl.ds(base, CH)], eid_v)
    ov = off_v[...]
    for k in range(CH // 16):
        ev = eid_v[pl.ds(k * 16, 16)]
        rv = rank_v[pl.ds(k * 16, 16)]
        og = lax.gather(
            ov, ev[:, None],
            dimension_numbers=lax.GatherDimensionNumbers(
                offset_dims=(), collapsed_slice_dims=(0,),
                start_index_map=(0,)),
            slice_sizes=(1,),
            mode=lax.GatherScatterMode.PROMISE_IN_BOUNDS)
        idx_v[pl.ds(k * 16, 16)] = rv + og


_SC_SCRATCH = [
    pltpu.VMEM((CH,), jnp.int32),
    pltpu.VMEM((CH,), jnp.int32),
    pltpu.VMEM((CH,), jnp.int32),
    pltpu.VMEM((16,), jnp.int32),
    pltpu.VMEM((CH, D), jnp.float32),
    pltpu.SemaphoreType.DMA,
]


@functools.partial(
    pl.kernel, mesh=_MESH,
    out_type=jax.ShapeDtypeStruct((NP, D), jnp.float32),
    scratch_types=_SC_SCRATCH,
)
def _sc_dispatch(x_hbm, rank_hbm, eid_hbm, off_hbm, xs_hbm,
                 rank_v, eid_v, idx_v, off_v, rows_v, sem):
    wid = lax.axis_index("s") * _NC + lax.axis_index("c")
    pltpu.sync_copy(off_hbm, off_v)
    for j in range(TPW // CH):
        base = wid * TPW + j * CH
        _slots_chunk(rank_hbm, eid_hbm, base, rank_v, eid_v, off_v, idx_v)
        pltpu.sync_copy(x_hbm.at[pl.ds(base, CH)], rows_v)
        pltpu.async_copy(rows_v, xs_hbm.at[idx_v], sem).wait()


@functools.partial(
    pl.kernel, mesh=_MESH,
    out_type=jax.ShapeDtypeStruct((N, D), jnp.float32),
    scratch_types=_SC_SCRATCH,
)
def _sc_combine(y_hbm, rank_hbm, eid_hbm, off_hbm, out_hbm,
                rank_v, eid_v, idx_v, off_v, rows_v, sem):
    wid = lax.axis_index("s") * _NC + lax.axis_index("c")
    pltpu.sync_copy(off_hbm, off_v)
    for j in range(TPW // CH):
        base = wid * TPW + j * CH
        _slots_chunk(rank_hbm, eid_hbm, base, rank_v, eid_v, off_v, idx_v)
        pltpu.async_copy(y_hbm.at[idx_v], rows_v, sem).wait()
        pltpu.sync_copy(rows_v, out_hbm.at[pl.ds(base, CH)])


def _erf(z):
    s = jnp.where(z >= 0.0, 1.0, -1.0)
    a = jnp.abs(z)
    t = 1.0 / (1.0 + 0.3275911 * a)
    poly = t * (0.254829592 + t * (-0.284496736 + t * (
        1.421413741 + t * (-1.453152027 + t * 1.061405429))))
    return s * (1.0 - poly * jnp.exp(-a * a))


def _gelu(v):
    return 0.5 * v * (1.0 + _erf(v * 0.7071067811865476))


FT = F // 2


FC = 512


def _ffn_chunks(x_ref, w1_ref, b1_ref, w2_ref):
    h = jnp.dot(x_ref[...], w1_ref[0], preferred_element_type=jnp.float32)
    g = _gelu(h + b1_ref[0])
    return jnp.dot(g, w2_ref[0], preferred_element_type=jnp.float32)


def _ffn_body_a(be_ref, x_ref, w1_ref, b1_ref, w2_ref, y_ref):
    y_ref[...] = _ffn_chunks(x_ref, w1_ref, b1_ref, w2_ref)


def _ffn_body_b(be_ref, x_ref, w1_ref, b1_ref, w2_ref, b2_ref, y0_ref, y_ref):
    y = _ffn_chunks(x_ref, w1_ref, b1_ref, w2_ref)
    y_ref[...] = y + y0_ref[...] + b2_ref[0]


def _run_ffn(be, xs, w1, b1, w2, b2):
    def specs(half, extra_y0):
        sp = [
            pl.BlockSpec((TM, D), lambda i, be: (i, 0)),
            pl.BlockSpec((1, D, FT), lambda i, be: (be[i], 0, half)),
            pl.BlockSpec((1, 1, FT), lambda i, be: (be[i], 0, half)),
            pl.BlockSpec((1, FT, D), lambda i, be: (be[i], half, 0)),
        ]
        if extra_y0:
            sp.append(pl.BlockSpec((1, 1, D), lambda i, be: (be[i], 0, 0)))
            sp.append(pl.BlockSpec((TM, D), lambda i, be: (i, 0)))
        return sp

    def call(body, half, args, extra_y0):
        grid_spec = pltpu.PrefetchScalarGridSpec(
            num_scalar_prefetch=1,
            grid=(NB,),
            in_specs=specs(half, extra_y0),
            out_specs=pl.BlockSpec((TM, D), lambda i, be: (i, 0)),
        )
        return pl.pallas_call(
            body,
            grid_spec=grid_spec,
            out_shape=jax.ShapeDtypeStruct((NP, D), jnp.float32),
        )(be, *args)

    y0 = call(_ffn_body_a, 0, (xs, w1, b1, w2), False)
    return call(_ffn_body_b, 1, (xs, w1, b1, w2, b2, y0), True)


def kernel(x, Wr, br, W1, b1, W2, b2):
    x2 = x.reshape(N, D)
    wrp = jnp.pad(Wr, ((0, 0), (0, LANES - E)))
    brp = jnp.pad(br.reshape(1, E), ((0, 0), (0, LANES - E)),
                  constant_values=-3.0e38)
    rank2, eid2, off2, be2 = _run_router(x2, wrp, brp)
    rank = rank2.reshape(N)
    eid = eid2.reshape(N)
    off16 = off2.reshape(LANES)[:16]
    be = be2.reshape(LANES)[:NB]
    xs = _sc_dispatch(x2, rank, eid, off16)
    y = _run_ffn(be, xs, W1, b1.reshape(E, 1, F), W2, b2.reshape(E, 1, D))
    out = _sc_combine(y, rank, eid, off16)
    return out.reshape(B, S, D)
